# baseline (speedup 1.0000x reference)
import functools
import jax
import jax.numpy as jnp
from jax import lax
from jax.experimental import pallas as pl
from jax.experimental.pallas import tpu as pltpu
from jax.experimental.pallas import tpu_sc as plsc

MAX_STEPS = 204.0
L = 16
HEAD = 16
TAIL = 16
U = 8
Q = 4


def _make_sc_shift(R, T, C):
    info = plsc.get_sparse_core_info()
    NC, NS = info.num_cores, info.num_subcores
    NW = NC * NS
    assert R % (2 * Q * NW) == 0 and C % L == 0
    nchunks = T // L
    n_int = nchunks - HEAD - TAIL
    assert T % L == 0 and n_int % U == 0 and HEAD * L >= MAX_STEPS + 1
    rows_per = R // NW
    pairs = rows_per // 2
    quads = pairs // Q

    mesh = plsc.VectorSubcoreMesh(core_axis_name="c", subcore_axis_name="s")

    @functools.partial(
        pl.kernel,
        mesh=mesh,
        out_type=jax.ShapeDtypeStruct((R * T,), jnp.float32),
        compiler_params=pltpu.CompilerParams(needs_layout_passes=False),
        scratch_types=[
            pltpu.VMEM((C,), jnp.float32),
            pltpu.VMEM((C,), jnp.int32),
            pltpu.VMEM((C,), jnp.float32),
            [pltpu.VMEM((2 * T,), jnp.float32) for _ in range(Q)],
            [pltpu.VMEM((2 * T,), jnp.float32) for _ in range(2)],
            [pltpu.SemaphoreType.DMA for _ in range(Q)],
            [pltpu.SemaphoreType.DMA for _ in range(2)],
        ],
    )
    def sc_shift(x_hbm, shift_hbm, out_hbm, shiftv, kbuf, abuf, ins, obs,
                 sis, sos):
        wid = lax.axis_index("s") * NC + lax.axis_index("c")
        pltpu.sync_copy(shift_hbm, shiftv)

        for i in range(C // L):
            p = shiftv[pl.ds(i * L, L)]
            e = jnp.exp(-2.0 * jnp.abs(p))
            s = jnp.sign(p) * ((1.0 - e) / (1.0 + e)) * MAX_STEPS
            tr = s.astype(jnp.int32)
            kf = jnp.where(tr.astype(jnp.float32) > s, tr - 1, tr)
            kbuf[pl.ds(i * L, L)] = kf
            abuf[pl.ds(i * L, L)] = s - kf.astype(jnp.float32)

        iota = lax.iota(jnp.int32, L)
        base_row = wid * rows_per

        def row_params(row, off):
            ch = jnp.full((L,), lax.rem(row, C), jnp.int32)
            av = plsc.load_gather(abuf, [ch])
            base = plsc.load_gather(kbuf, [ch]) + (iota + off)
            return av, 1.0 - av, base

        def clip_chunk(inb, ob, av, bv, base, t, lo, hi, oo):
            t = pl.multiple_of(t, L)
            idx = base + t
            i0 = jnp.minimum(jnp.maximum(idx, lo), hi)
            i1 = jnp.minimum(jnp.maximum(idx + 1, lo), hi)
            v0 = plsc.load_gather(inb, [i0])
            v1 = plsc.load_gather(inb, [i1])
            ob[pl.ds(oo + t, L)] = bv * v0 + av * v1

        def compute_row(inb, ob, av, bv, base, lo, hi, oo):
            @plsc.parallel_loop(0, HEAD, unroll=8)
            def _(ci):
                clip_chunk(inb, ob, av, bv, base, ci * L, lo, hi, oo)

            @plsc.parallel_loop(HEAD, nchunks - TAIL, unroll=U)
            def _(ci):
                t = pl.multiple_of(ci * L, L)
                i0 = base + t
                v0 = plsc.load_gather(inb, [i0])
                v1 = plsc.load_gather(inb, [i0 + 1])
                ob[pl.ds(oo + t, L)] = bv * v0 + av * v1

            @plsc.parallel_loop(nchunks - TAIL, nchunks, unroll=8)
            def _(ci):
                clip_chunk(inb, ob, av, bv, base, ci * L, lo, hi, oo)

        def in_slice(jp):
            return x_hbm.at[pl.ds(
                pl.multiple_of((base_row + 2 * jp) * T, 8), 2 * T)]

        def out_slice(jp):
            return out_hbm.at[pl.ds(
                pl.multiple_of((base_row + 2 * jp) * T, 8), 2 * T)]

        for u in range(Q - 1):
            pltpu.make_async_copy(in_slice(u), ins[u], sis[u]).start()

        def quad_body(q, _):
            for u in range(Q):
                jp = Q * q + u
                nxt = (u + Q - 1) % Q

                @pl.when(jp + Q - 1 < pairs)
                def _():
                    pltpu.make_async_copy(
                        in_slice(jp + Q - 1), ins[nxt], sis[nxt]).start()

                pltpu.make_async_copy(in_slice(jp), ins[u], sis[u]).wait()
                r0 = base_row + 2 * jp

                @pl.when(jp >= 2)
                def _():
                    pltpu.make_async_copy(
                        obs[u % 2], out_slice(jp), sos[u % 2]).wait()

                av0, bv0, base0 = row_params(r0, 0)
                compute_row(ins[u], obs[u % 2], av0, bv0, base0,
                            0, T - 1, 0)
                av1, bv1, base1 = row_params(r0 + 1, T)
                compute_row(ins[u], obs[u % 2], av1, bv1, base1,
                            T, 2 * T - 1, T)
                pltpu.make_async_copy(
                    obs[u % 2], out_slice(jp), sos[u % 2]).start()
            return 0

        lax.fori_loop(0, quads, quad_body, 0)
        for u in range(2):
            pltpu.make_async_copy(obs[u], out_slice(u), sos[u]).wait()

    return sc_shift


def kernel(x, shift_param):
    B, C, T = x.shape
    xr = x.reshape(B * C * T)
    sp = shift_param.reshape(C).astype(jnp.float32)
    out = _make_sc_shift(B * C, T, C)(xr, sp)
    return out.reshape(B, C, T)


# speedup vs baseline: 2.9591x; 2.9591x over previous
import functools
import jax
import jax.numpy as jnp
from jax import lax
from jax.experimental import pallas as pl
from jax.experimental.pallas import tpu as pltpu
from jax.experimental.pallas import tpu_sc as plsc

MAX_STEPS = 204.0
L = 16
HEAD = 16
TAIL = 16
U = 8
Q = 8
QO = 4


def _make_sc_shift(R, T, C):
    info = plsc.get_sparse_core_info()
    NC, NS = info.num_cores, info.num_subcores
    NW = NC * NS
    assert R % (Q * NW) == 0 and C % L == 0
    nchunks = T // L
    n_int = nchunks - HEAD - TAIL
    assert T % L == 0 and n_int % U == 0 and HEAD * L >= MAX_STEPS + 1
    rows_per = R // NW
    quads = rows_per // Q

    mesh = plsc.VectorSubcoreMesh(core_axis_name="c", subcore_axis_name="s")

    @functools.partial(
        pl.kernel,
        mesh=mesh,
        out_type=jax.ShapeDtypeStruct((R, T), jnp.float32),
        compiler_params=pltpu.CompilerParams(needs_layout_passes=False),
        scratch_types=[
            pltpu.VMEM((C,), jnp.float32),
            pltpu.VMEM((C,), jnp.int32),
            pltpu.VMEM((C,), jnp.float32),
            [pltpu.VMEM((T,), jnp.float32) for _ in range(Q)],
            [pltpu.VMEM((T,), jnp.float32) for _ in range(QO)],
            [pltpu.SemaphoreType.DMA for _ in range(Q)],
            [pltpu.SemaphoreType.DMA for _ in range(QO)],
        ],
    )
    def sc_shift(x_hbm, shift_hbm, out_hbm, shiftv, kbuf, abuf, ins, obs,
                 sis, sos):
        wid = lax.axis_index("s") * NC + lax.axis_index("c")
        pltpu.sync_copy(shift_hbm, shiftv)

        for i in range(C // L):
            p = shiftv[pl.ds(i * L, L)]
            e = jnp.exp(-2.0 * jnp.abs(p))
            s = jnp.sign(p) * ((1.0 - e) / (1.0 + e)) * MAX_STEPS
            tr = s.astype(jnp.int32)
            kf = jnp.where(tr.astype(jnp.float32) > s, tr - 1, tr)
            kbuf[pl.ds(i * L, L)] = kf
            abuf[pl.ds(i * L, L)] = s - kf.astype(jnp.float32)

        iota = lax.iota(jnp.int32, L)
        base_row = wid * rows_per

        def row_params(row):
            ch = jnp.full((L,), lax.rem(row, C), jnp.int32)
            av = plsc.load_gather(abuf, [ch])
            base = plsc.load_gather(kbuf, [ch]) + iota
            return av, 1.0 - av, base

        def clip_chunk(inb, ob, av, bv, base, t):
            t = pl.multiple_of(t, L)
            idx = base + t
            i0 = jnp.minimum(jnp.maximum(idx, 0), T - 1)
            i1 = jnp.minimum(jnp.maximum(idx + 1, 0), T - 1)
            v0 = plsc.load_gather(inb, [i0])
            v1 = plsc.load_gather(inb, [i1])
            ob[pl.ds(t, L)] = bv * v0 + av * v1

        def compute_row(inb, ob, av, bv, base):
            @plsc.parallel_loop(0, HEAD, unroll=8)
            def _(ci):
                clip_chunk(inb, ob, av, bv, base, ci * L)

            @plsc.parallel_loop(HEAD, nchunks - TAIL, unroll=U)
            def _(ci):
                t = pl.multiple_of(ci * L, L)
                i0 = base + t
                v0 = plsc.load_gather(inb, [i0])
                v1 = plsc.load_gather(inb, [i0 + 1])
                ob[pl.ds(t, L)] = bv * v0 + av * v1

            @plsc.parallel_loop(nchunks - TAIL, nchunks, unroll=8)
            def _(ci):
                clip_chunk(inb, ob, av, bv, base, ci * L)

        for u in range(Q - 1):
            pltpu.make_async_copy(x_hbm.at[base_row + u], ins[u], sis[u]).start()

        def quad_body(q, _):
            r0 = base_row + Q * q
            j0 = Q * q
            for u in range(Q):
                r = r0 + u
                nxt = (u + Q - 1) % Q

                @pl.when(j0 + u + Q - 1 < rows_per)
                def _():
                    pltpu.make_async_copy(
                        x_hbm.at[r + Q - 1], ins[nxt], sis[nxt]).start()

                pltpu.make_async_copy(x_hbm.at[r], ins[u], sis[u]).wait()
                av, bv, base = row_params(r)

                @pl.when(j0 + u >= QO)
                def _():
                    pltpu.make_async_copy(
                        obs[u % QO], out_hbm.at[r], sos[u % QO]).wait()

                compute_row(ins[u], obs[u % QO], av, bv, base)
                pltpu.make_async_copy(
                    obs[u % QO], out_hbm.at[r], sos[u % QO]).start()
            return 0

        lax.fori_loop(0, quads, quad_body, 0)
        for u in range(QO):
            pltpu.make_async_copy(
                obs[u], out_hbm.at[base_row + u], sos[u]).wait()

    return sc_shift


def kernel(x, shift_param):
    B, C, T = x.shape
    xr = x.reshape(B * C, T)
    sp = shift_param.reshape(C).astype(jnp.float32)
    out = _make_sc_shift(B * C, T, C)(xr, sp)
    return out.reshape(B, C, T)


# speedup vs baseline: 3.0136x; 1.0184x over previous
import functools
import jax
import jax.numpy as jnp
from jax import lax
from jax.experimental import pallas as pl
from jax.experimental.pallas import tpu as pltpu
from jax.experimental.pallas import tpu_sc as plsc

MAX_STEPS = 204.0
L = 16
HEAD = 16
TAIL = 16
U = 8
Q = 4


def _make_sc_shift(R, T, C):
    info = plsc.get_sparse_core_info()
    NC, NS = info.num_cores, info.num_subcores
    NW = NC * NS
    assert R % (Q * NW) == 0 and C % L == 0
    nchunks = T // L
    n_int = nchunks - HEAD - TAIL
    assert T % L == 0 and n_int % U == 0 and HEAD * L >= MAX_STEPS + 1
    rows_per = R // NW
    quads = rows_per // Q

    mesh = plsc.VectorSubcoreMesh(core_axis_name="c", subcore_axis_name="s")

    @functools.partial(
        pl.kernel,
        mesh=mesh,
        out_type=jax.ShapeDtypeStruct((R, T), jnp.float32),
        compiler_params=pltpu.CompilerParams(needs_layout_passes=False),
        scratch_types=[
            pltpu.VMEM((C,), jnp.float32),
            pltpu.VMEM((C,), jnp.int32),
            pltpu.VMEM((C,), jnp.float32),
            [pltpu.VMEM((T,), jnp.float32) for _ in range(Q)],
            [pltpu.VMEM((T,), jnp.float32) for _ in range(Q)],
            [pltpu.SemaphoreType.DMA for _ in range(Q)],
            [pltpu.SemaphoreType.DMA for _ in range(Q)],
        ],
    )
    def sc_shift(x_hbm, shift_hbm, out_hbm, shiftv, kbuf, abuf, ins, obs,
                 sis, sos):
        wid = lax.axis_index("s") * NC + lax.axis_index("c")
        pltpu.sync_copy(shift_hbm, shiftv)

        for i in range(C // L):
            p = shiftv[pl.ds(i * L, L)]
            e = jnp.exp(-2.0 * jnp.abs(p))
            s = jnp.sign(p) * ((1.0 - e) / (1.0 + e)) * MAX_STEPS
            tr = s.astype(jnp.int32)
            kf = jnp.where(tr.astype(jnp.float32) > s, tr - 1, tr)
            kbuf[pl.ds(i * L, L)] = kf
            abuf[pl.ds(i * L, L)] = s - kf.astype(jnp.float32)

        iota = lax.iota(jnp.int32, L)
        base_row = wid * rows_per

        def row_params(row):
            ch = jnp.full((L,), lax.rem(row, C), jnp.int32)
            av = plsc.load_gather(abuf, [ch])
            base = plsc.load_gather(kbuf, [ch]) + iota
            return av, 1.0 - av, base

        def clip_chunk(inb, ob, av, bv, base, t):
            t = pl.multiple_of(t, L)
            idx = base + t
            i0 = jnp.minimum(jnp.maximum(idx, 0), T - 1)
            i1 = jnp.minimum(jnp.maximum(idx + 1, 0), T - 1)
            v0 = plsc.load_gather(inb, [i0])
            v1 = plsc.load_gather(inb, [i1])
            ob[pl.ds(t, L)] = bv * v0 + av * v1

        def compute_row(inb, ob, av, bv, base):
            @plsc.parallel_loop(0, HEAD, unroll=8)
            def _(ci):
                clip_chunk(inb, ob, av, bv, base, ci * L)

            @plsc.parallel_loop(HEAD, nchunks - TAIL, unroll=U)
            def _(ci):
                t = pl.multiple_of(ci * L, L)
                i0 = base + t
                v0 = plsc.load_gather(inb, [i0])
                v1 = plsc.load_gather(inb, [i0 + 1])
                ob[pl.ds(t, L)] = bv * v0 + av * v1

            @plsc.parallel_loop(nchunks - TAIL, nchunks, unroll=8)
            def _(ci):
                clip_chunk(inb, ob, av, bv, base, ci * L)

        for u in range(Q - 1):
            pltpu.make_async_copy(x_hbm.at[base_row + u], ins[u], sis[u]).start()

        def quad_body(q, _):
            r0 = base_row + Q * q
            j0 = Q * q
            for u in range(Q):
                r = r0 + u
                nxt = (u + Q - 1) % Q

                @pl.when(j0 + u + Q - 1 < rows_per)
                def _():
                    pltpu.make_async_copy(
                        x_hbm.at[r + Q - 1], ins[nxt], sis[nxt]).start()

                pltpu.make_async_copy(x_hbm.at[r], ins[u], sis[u]).wait()
                av, bv, base = row_params(r)

                @pl.when(j0 + u >= Q)
                def _():
                    pltpu.make_async_copy(
                        obs[u], out_hbm.at[r], sos[u]).wait()

                compute_row(ins[u], obs[u], av, bv, base)
                pltpu.make_async_copy(
                    obs[u], out_hbm.at[r], sos[u]).start()
            return 0

        lax.fori_loop(0, quads, quad_body, 0)
        for u in range(Q):
            pltpu.make_async_copy(
                obs[u], out_hbm.at[base_row + u], sos[u]).wait()

    return sc_shift


def kernel(x, shift_param):
    B, C, T = x.shape
    xr = x.reshape(B * C, T)
    sp = shift_param.reshape(C).astype(jnp.float32)
    out = _make_sc_shift(B * C, T, C)(xr, sp)
    return out.reshape(B, C, T)
